# baseline (speedup 1.0000x reference)
import functools

import jax
import jax.numpy as jnp
import numpy as np
from jax.experimental import pallas as pl
from jax.experimental.pallas import tpu as pltpu

_BATCH = 8
_NP = 5000
_NPP = 5120
_CHUNK = 1280
_NCH = _NPP // _CHUNK
_MAXGT = 100
_GTL = 128
_GTR = 104
_MH = 56
_MW = 56
_MKH = 28
_MKW = 28
_TRAIN = 200
_POS = 66
_NEG = 134
_POSP = 72
_BBOX_STD = np.array([0.1, 0.1, 0.2, 0.2], dtype=np.float32)


def _body(p_ref, gtt_ref, gtb_ref, cls_ref, msk_ref,
          rois_ref, clso_ref, dlt_ref, msko_ref,
          ov_scr, spos_scr, sneg_scr, negb_scr,
          rrois_scr, orows_scr, validv_scr, sr_scr, sg_scr):
    f32 = jnp.float32

    clso_ref[0] = jnp.zeros((_TRAIN, 1), jnp.int32)
    dlt_ref[0] = jnp.zeros((_TRAIN, 4), f32)
    msko_ref[0] = jnp.zeros((_TRAIN, _MKH, _MKW), f32)

    gy1 = gtt_ref[0, 0:1, :]
    gx1 = gtt_ref[0, 1:2, :]
    gy2 = gtt_ref[0, 2:3, :]
    gx2 = gtt_ref[0, 3:4, :]
    cls_row = cls_ref[0, 0:1, :]
    gt_valid = (jnp.abs(gy1) + jnp.abs(gx1) + jnp.abs(gy2) + jnp.abs(gx2)) > 0.0
    crowd = (cls_row < 0.0) & gt_valid
    non_crowd = gt_valid & (cls_row >= 0.0)
    area2 = (gy2 - gy1) * (gx2 - gx1)

    for c in range(_NCH):
        r0 = c * _CHUNK
        pc = p_ref[0, pl.ds(r0, _CHUNK), :]
        py1 = pc[:, 0:1]
        px1 = pc[:, 1:2]
        py2 = pc[:, 2:3]
        px2 = pc[:, 3:4]
        prop_valid = (jnp.abs(py1) + jnp.abs(px1) +
                      jnp.abs(py2) + jnp.abs(px2)) > 0.0
        y1 = jnp.maximum(py1, gy1)
        x1 = jnp.maximum(px1, gx1)
        y2 = jnp.minimum(py2, gy2)
        x2 = jnp.minimum(px2, gx2)
        inter = jnp.maximum(y2 - y1, 0.0) * jnp.maximum(x2 - x1, 0.0)
        area1 = (py2 - py1) * (px2 - px1)
        union = area1 + area2 - inter
        iou = inter / jnp.maximum(union, 1e-8)
        ov_nc = jnp.where(non_crowd, iou, -1.0)
        ov_scr[pl.ds(r0, _CHUNK), :] = ov_nc
        crowd_ov = jnp.where(crowd, iou, -1.0)
        crowd_max = jnp.max(crowd_ov, axis=1, keepdims=True)
        no_crowd = crowd_max < 0.001
        rim = jnp.max(ov_nc, axis=1, keepdims=True)
        positive = (rim >= 0.5) & prop_valid
        negative = (rim < 0.5) & no_crowd & prop_valid
        pos_s = jnp.where(positive, rim, -1.0)
        neg_s = jnp.where(negative, rim, -1.0)
        spos_scr[pl.ds(c * 10, 10), :] = pos_s.reshape(10, 128)
        sneg_scr[pl.ds(c * 10, 10), :] = neg_s.reshape(10, 128)
        negb_scr[pl.ds(r0, _CHUNK), :] = negative.astype(f32)

    iota_f = (jax.lax.broadcasted_iota(jnp.int32, (40, 128), 0) * 128 +
              jax.lax.broadcasted_iota(jnp.int32, (40, 128), 1)).astype(f32)

    def pos_step(k, carry):
        s = spos_scr[...]
        m = jnp.max(s)
        idxf = jnp.min(jnp.where(s == m, iota_f, 1e9))
        spos_scr[...] = jnp.where(iota_f == idxf, -3.0, s)
        idx = idxf.astype(jnp.int32)
        v = (m >= 0.5).astype(f32).reshape(1, 1)
        rrois_scr[pl.ds(k, 1), :] = p_ref[0, pl.ds(idx, 1), :]
        orows_scr[pl.ds(k, 1), :] = ov_scr[pl.ds(idx, 1), :]
        validv_scr[pl.ds(k, 1), :] = v
        return carry

    jax.lax.fori_loop(0, _POS, pos_step, 0)

    def neg_step(k, carry):
        s = sneg_scr[...]
        m = jnp.max(s)
        idxf = jnp.min(jnp.where(s == m, iota_f, 1e9))
        sneg_scr[...] = jnp.where(iota_f == idxf, -3.0, s)
        idx = idxf.astype(jnp.int32)
        rrois_scr[pl.ds(_POS + k, 1), :] = p_ref[0, pl.ds(idx, 1), :]
        validv_scr[pl.ds(_POS + k, 1), :] = negb_scr[pl.ds(idx, 1), :]
        return carry

    jax.lax.fori_loop(0, _NEG, neg_step, 0)

    rois_ref[0] = rrois_scr[...] * validv_scr[...]

    safe_full = (jax.lax.broadcasted_iota(jnp.int32, (_POSP, 4), 1) >= 2
                 ).astype(f32)
    sr_scr[...] = safe_full
    sg_scr[...] = safe_full
    safe_row = (jax.lax.broadcasted_iota(jnp.int32, (1, 4), 1) >= 2).astype(f32)

    lane128 = jax.lax.broadcasted_iota(jnp.int32, (1, 128), 1)
    gy28 = jax.lax.broadcasted_iota(jnp.int32, (_MKH, 1), 0).astype(f32) / (_MKH - 1.0)
    gx28 = jax.lax.broadcasted_iota(jnp.int32, (1, _MKW), 1).astype(f32) / (_MKW - 1.0)
    irow56 = jax.lax.broadcasted_iota(jnp.int32, (1, _MH), 1).astype(f32)
    icol56 = jax.lax.broadcasted_iota(jnp.int32, (_MW, 1), 0).astype(f32)

    def mask_step(k, carry):
        vv = validv_scr[pl.ds(k, 1), :]
        orow = orows_scr[pl.ds(k, 1), :]
        om = jnp.max(orow)
        g = jnp.min(jnp.where(orow == om, lane128, 128)).astype(jnp.int32)
        cvec = jnp.sum(jnp.where(lane128 == g, cls_row, 0.0),
                       axis=1, keepdims=True)
        clso_ref[0, pl.ds(k, 1), :] = (cvec * vv).astype(jnp.int32)
        prow = rrois_scr[pl.ds(k, 1), :]
        grow = gtb_ref[0, pl.ds(g, 1), :]
        srow = jnp.where(vv > 0.0, prow, safe_row)
        sgrow = jnp.where(vv > 0.0, grow, safe_row)
        sr_scr[pl.ds(k, 1), :] = srow
        sg_scr[pl.ds(k, 1), :] = sgrow
        sy1 = srow[:, 0:1]
        sx1 = srow[:, 1:2]
        sy2 = srow[:, 2:3]
        sx2 = srow[:, 3:4]
        ty1 = sgrow[:, 0:1]
        tx1 = sgrow[:, 1:2]
        ty2 = sgrow[:, 2:3]
        tx2 = sgrow[:, 3:4]
        gh = jnp.maximum(ty2 - ty1, 1e-8)
        gw = jnp.maximum(tx2 - tx1, 1e-8)
        by1 = (sy1 - ty1) / gh
        bx1 = (sx1 - tx1) / gw
        by2 = (sy2 - ty1) / gh
        bx2 = (sx2 - tx1) / gw
        ys = by1 * (_MH - 1.0) + gy28 * ((by2 - by1) * (_MH - 1.0))
        xs = bx1 * (_MW - 1.0) + gx28 * ((bx2 - bx1) * (_MW - 1.0))
        y0 = jnp.floor(ys)
        wy = ys - y0
        x0 = jnp.floor(xs)
        wx = xs - x0
        w_y = ((irow56 == y0).astype(f32) * (1.0 - wy) +
               (irow56 == y0 + 1.0).astype(f32) * wy)
        w_xt = ((icol56 == x0).astype(f32) * (1.0 - wx) +
                (icol56 == x0 + 1.0).astype(f32) * wx)
        mrow = msk_ref[0, pl.ds(g, 1), :, :][0]
        t = jnp.dot(mrow, w_xt, preferred_element_type=f32,
                    precision=jax.lax.Precision.HIGHEST)
        crop = jnp.dot(w_y, t, preferred_element_type=f32,
                       precision=jax.lax.Precision.HIGHEST)
        out = jnp.round(crop) * vv
        msko_ref[0, pl.ds(k, 1), :, :] = out.reshape(1, _MKH, _MKW)
        return carry

    jax.lax.fori_loop(0, _POS, mask_step, 0)

    sr = sr_scr[...]
    sg = sg_scr[...]
    h = sr[:, 2:3] - sr[:, 0:1]
    w = sr[:, 3:4] - sr[:, 1:2]
    cy = sr[:, 0:1] + 0.5 * h
    cx = sr[:, 1:2] + 0.5 * w
    gh = sg[:, 2:3] - sg[:, 0:1]
    gw = sg[:, 3:4] - sg[:, 1:2]
    gcy = sg[:, 0:1] + 0.5 * gh
    gcx = sg[:, 1:2] + 0.5 * gw
    dy = (gcy - cy) / h
    dx = (gcx - cx) / w
    dh = jnp.log(gh / h)
    dw = jnp.log(gw / w)
    std = jnp.where(
        jax.lax.broadcasted_iota(jnp.int32, (1, 4), 1) < 2, 0.1, 0.2)
    dl = jnp.concatenate([dy, dx, dh, dw], axis=1) / std
    rowp = jax.lax.broadcasted_iota(jnp.int32, (_POSP, 1), 0)
    pm = jnp.where(rowp < _POS, validv_scr[pl.ds(0, _POSP), :], 0.0)
    dlt_ref[0, pl.ds(0, _POSP), :] = dl * pm


@jax.jit
def kernel(proposals, prior_class_ids, prior_boxes, prior_masks):
    b = proposals.shape[0]
    f32 = jnp.float32
    p_pad = jnp.pad(proposals, ((0, 0), (0, _NPP - _NP), (0, 0)))
    gt_t = jnp.pad(jnp.transpose(prior_boxes, (0, 2, 1)),
                   ((0, 0), (0, 0), (0, _GTL - _MAXGT)))
    gt_r = jnp.pad(prior_boxes, ((0, 0), (0, _GTR - _MAXGT), (0, 0)))
    cls_f = jnp.pad(prior_class_ids.astype(f32),
                    ((0, 0), (0, _GTL - _MAXGT)))[:, None, :]
    msk_t = jnp.pad(jnp.transpose(prior_masks, (0, 3, 1, 2)),
                    ((0, 0), (0, _GTR - _MAXGT), (0, 0), (0, 0)))

    grid = (b,)
    rois, cls_o, deltas, masks = pl.pallas_call(
        _body,
        grid=grid,
        in_specs=[
            pl.BlockSpec((1, _NPP, 4), lambda i: (i, 0, 0)),
            pl.BlockSpec((1, 4, _GTL), lambda i: (i, 0, 0)),
            pl.BlockSpec((1, _GTR, 4), lambda i: (i, 0, 0)),
            pl.BlockSpec((1, 1, _GTL), lambda i: (i, 0, 0)),
            pl.BlockSpec((1, _GTR, _MH, _MW), lambda i: (i, 0, 0, 0)),
        ],
        out_specs=[
            pl.BlockSpec((1, _TRAIN, 4), lambda i: (i, 0, 0)),
            pl.BlockSpec((1, _TRAIN, 1), lambda i: (i, 0, 0)),
            pl.BlockSpec((1, _TRAIN, 4), lambda i: (i, 0, 0)),
            pl.BlockSpec((1, _TRAIN, _MKH, _MKW), lambda i: (i, 0, 0, 0)),
        ],
        out_shape=[
            jax.ShapeDtypeStruct((b, _TRAIN, 4), f32),
            jax.ShapeDtypeStruct((b, _TRAIN, 1), jnp.int32),
            jax.ShapeDtypeStruct((b, _TRAIN, 4), f32),
            jax.ShapeDtypeStruct((b, _TRAIN, _MKH, _MKW), f32),
        ],
        scratch_shapes=[
            pltpu.VMEM((_NPP, _GTL), f32),
            pltpu.VMEM((40, 128), f32),
            pltpu.VMEM((40, 128), f32),
            pltpu.VMEM((_NPP, 1), f32),
            pltpu.VMEM((_TRAIN, 4), f32),
            pltpu.VMEM((_POSP, _GTL), f32),
            pltpu.VMEM((_TRAIN, 1), f32),
            pltpu.VMEM((_POSP, 4), f32),
            pltpu.VMEM((_POSP, 4), f32),
        ],
    )(p_pad, gt_t, gt_r, cls_f, msk_t)
    return rois, cls_o[:, :, 0], deltas, masks


# speedup vs baseline: 1.8507x; 1.8507x over previous
import jax
import jax.numpy as jnp
from jax.experimental import pallas as pl
from jax.experimental.pallas import tpu as pltpu

_NP = 5000
_NPP = 5120
_CHUNK = 1280
_NCH = _NPP // _CHUNK
_MAXGT = 100
_GTL = 128
_MH = 56
_MW = 56
_MKH = 28
_MKW = 28
_TRAIN = 200
_POS = 66
_NEG = 134
_POSP = 72


def _body(p_ref, gtt_ref, gtb_ref, cls_ref, msk_ref,
          rois_ref, clso_ref, dlt_ref, msko_ref,
          spos_scr, sneg_scr, negb_scr, rrois_scr, validv_scr,
          g_scr, box_scr):
    f32 = jnp.float32
    nb = gtt_ref.shape[0]
    lane128 = jax.lax.broadcasted_iota(jnp.int32, (1, 128), 1)

    def gt_rows(b):
        gy1 = gtt_ref[b, 0:1, :]
        gx1 = gtt_ref[b, 1:2, :]
        gy2 = gtt_ref[b, 2:3, :]
        gx2 = gtt_ref[b, 3:4, :]
        cls_row = cls_ref[b, 0:1, :]
        gt_valid = (jnp.abs(gy1) + jnp.abs(gx1) +
                    jnp.abs(gy2) + jnp.abs(gx2)) > 0.0
        crowd = (cls_row < 0.0) & gt_valid
        non_crowd = gt_valid & (cls_row >= 0.0)
        return gy1, gx1, gy2, gx2, cls_row, crowd, non_crowd

    def iou_rows(pc, gt):
        gy1, gx1, gy2, gx2, _, _, non_crowd = gt
        py1 = pc[:, 0:1]
        px1 = pc[:, 1:2]
        py2 = pc[:, 2:3]
        px2 = pc[:, 3:4]
        y1 = jnp.maximum(py1, gy1)
        x1 = jnp.maximum(px1, gx1)
        y2 = jnp.minimum(py2, gy2)
        x2 = jnp.minimum(px2, gx2)
        inter = jnp.maximum(y2 - y1, 0.0) * jnp.maximum(x2 - x1, 0.0)
        area1 = (py2 - py1) * (px2 - px1)
        area2 = (gy2 - gy1) * (gx2 - gx1)
        union = area1 + area2 - inter
        iou = inter / jnp.maximum(union, 1e-8)
        return iou, jnp.where(non_crowd, iou, -1.0)

    msko_ref[...] = jnp.zeros((_TRAIN * 32, nb * _MKW), f32)
    for b in range(nb):
        clso_ref[b] = jnp.zeros((_TRAIN, 1), jnp.int32)
        dlt_ref[b] = jnp.zeros((_TRAIN, 4), f32)

    for b in range(nb):
        gt = gt_rows(b)
        crowd = gt[5]
        for c in range(_NCH):
            r0 = c * _CHUNK
            pc = p_ref[pl.ds(r0, _CHUNK), b * 4:(b + 1) * 4]
            prop_valid = (jnp.abs(pc[:, 0:1]) + jnp.abs(pc[:, 1:2]) +
                          jnp.abs(pc[:, 2:3]) + jnp.abs(pc[:, 3:4])) > 0.0
            iou, ov_nc = iou_rows(pc, gt)
            crowd_ov = jnp.where(crowd, iou, -1.0)
            no_crowd = jnp.max(crowd_ov, axis=1, keepdims=True) < 0.001
            rim = jnp.max(ov_nc, axis=1, keepdims=True)
            positive = (rim >= 0.5) & prop_valid
            negative = (rim < 0.5) & no_crowd & prop_valid
            pos_s = jnp.where(positive, rim, -1.0)
            neg_s = jnp.where(negative, rim, -1.0)
            spos_scr[pl.ds(b * 40 + c * 10, 10), :] = pos_s.reshape(10, 128)
            sneg_scr[pl.ds(b * 40 + c * 10, 10), :] = neg_s.reshape(10, 128)
            negb_scr[pl.ds(r0, _CHUNK), b:b + 1] = negative.astype(f32)

    iota_f = (jax.lax.broadcasted_iota(jnp.int32, (40, 128), 0) * 128 +
              jax.lax.broadcasted_iota(jnp.int32, (40, 128), 1)).astype(f32)

    def pos_one(b, k):
        s = spos_scr[pl.ds(b * 40, 40), :]
        m = jnp.max(s)
        idxf = jnp.min(jnp.where(s == m, iota_f, 1e9))
        spos_scr[pl.ds(b * 40, 40), :] = jnp.where(iota_f == idxf, -3.0, s)
        idx = idxf.astype(jnp.int32)
        rrois_scr[pl.ds(b * _TRAIN + k, 1), :] = \
            p_ref[pl.ds(idx, 1), b * 4:(b + 1) * 4]
        validv_scr[pl.ds(b * _TRAIN + k, 1), :] = \
            (m >= 0.5).astype(f32).reshape(1, 1)

    def neg_one(b, k):
        s = sneg_scr[pl.ds(b * 40, 40), :]
        m = jnp.max(s)
        idxf = jnp.min(jnp.where(s == m, iota_f, 1e9))
        sneg_scr[pl.ds(b * 40, 40), :] = jnp.where(iota_f == idxf, -3.0, s)
        idx = idxf.astype(jnp.int32)
        rrois_scr[pl.ds(b * _TRAIN + _POS + k, 1), :] = \
            p_ref[pl.ds(idx, 1), b * 4:(b + 1) * 4]
        validv_scr[pl.ds(b * _TRAIN + _POS + k, 1), :] = \
            negb_scr[pl.ds(idx, 1), b:b + 1]

    def both_step(k, carry):
        for b in range(nb):
            pos_one(b, k)
            neg_one(b, k)
        return carry

    def neg_step(k, carry):
        for b in range(nb):
            neg_one(b, k)
        return carry

    jax.lax.fori_loop(0, _POS, both_step, 0)
    jax.lax.fori_loop(_POS, _NEG, neg_step, 0)

    for b in range(nb):
        rois_ref[b] = (rrois_scr[pl.ds(b * _TRAIN, _TRAIN), :] *
                       validv_scr[pl.ds(b * _TRAIN, _TRAIN), :])

    safe_full = (jax.lax.broadcasted_iota(jnp.int32, (_POSP, 4), 1) >= 2
                 ).astype(f32)
    row72 = jax.lax.broadcasted_iota(jnp.int32, (_POSP, 1), 0)
    std = jnp.where(
        jax.lax.broadcasted_iota(jnp.int32, (1, 4), 1) < 2, 0.1, 0.2)
    for b in range(nb):
        gt = gt_rows(b)
        cls_row = gt[4]
        rp = rrois_scr[pl.ds(b * _TRAIN, _POSP), :]
        vp = validv_scr[pl.ds(b * _TRAIN, _POSP), :]
        _, ov72 = iou_rows(rp, gt)
        gm = jnp.max(ov72, axis=1, keepdims=True)
        g72 = jnp.min(jnp.where(ov72 == gm, lane128, 128),
                      axis=1, keepdims=True)
        g_scr[pl.ds(b * _POSP, _POSP), :] = g72.astype(f32)
        cond = (vp > 0.0) & (row72 < _POS)
        pm = cond.astype(f32)
        csel = jnp.sum(jnp.where(lane128 == g72, cls_row, 0.0),
                       axis=1, keepdims=True)
        clso_ref[b, pl.ds(0, _POSP), :] = (csel * pm).astype(jnp.int32)
        onehot = (lane128 == g72).astype(f32)
        gtsel = jnp.dot(onehot, gtb_ref[b], preferred_element_type=f32,
                        precision=jax.lax.Precision.HIGHEST)
        sr = jnp.where(cond, rp, safe_full)
        sg = jnp.where(cond, gtsel, safe_full)
        h = sr[:, 2:3] - sr[:, 0:1]
        w = sr[:, 3:4] - sr[:, 1:2]
        cy = sr[:, 0:1] + 0.5 * h
        cx = sr[:, 1:2] + 0.5 * w
        gh = sg[:, 2:3] - sg[:, 0:1]
        gw = sg[:, 3:4] - sg[:, 1:2]
        gcy = sg[:, 0:1] + 0.5 * gh
        gcx = sg[:, 1:2] + 0.5 * gw
        dl = jnp.concatenate(
            [(gcy - cy) / h, (gcx - cx) / w,
             jnp.log(gh / h), jnp.log(gw / w)], axis=1) / std
        dlt_ref[b, pl.ds(0, _POSP), :] = dl * pm
        ghc = jnp.maximum(sg[:, 2:3] - sg[:, 0:1], 1e-8)
        gwc = jnp.maximum(sg[:, 3:4] - sg[:, 1:2], 1e-8)
        by1 = (sr[:, 0:1] - sg[:, 0:1]) / ghc
        bx1 = (sr[:, 1:2] - sg[:, 1:2]) / gwc
        by2 = (sr[:, 2:3] - sg[:, 0:1]) / ghc
        bx2 = (sr[:, 3:4] - sg[:, 1:2]) / gwc
        box_scr[pl.ds(b * _POSP, _POSP), :] = jnp.concatenate(
            [by1, bx1, by2, bx2], axis=1)

    gy28 = jax.lax.broadcasted_iota(jnp.int32, (_MKH, 1), 0).astype(f32) \
        / (_MKH - 1.0)
    gx28 = jax.lax.broadcasted_iota(jnp.int32, (1, _MKW), 1).astype(f32) \
        / (_MKW - 1.0)
    irow56 = jax.lax.broadcasted_iota(jnp.int32, (1, _MH), 1).astype(f32)
    icol56 = jax.lax.broadcasted_iota(jnp.int32, (_MW, 1), 0).astype(f32)

    def mask_step(k, carry):
        for b in range(nb):
            vv = validv_scr[pl.ds(b * _TRAIN + k, 1), :]
            garr = g_scr[pl.ds(b * _POSP + k, 1), :]
            g = garr[0, 0].astype(jnp.int32)
            bx = box_scr[pl.ds(b * _POSP + k, 1), :]
            by1 = bx[:, 0:1]
            bx1 = bx[:, 1:2]
            by2 = bx[:, 2:3]
            bx2 = bx[:, 3:4]
            ys = by1 * (_MH - 1.0) + gy28 * ((by2 - by1) * (_MH - 1.0))
            xs = bx1 * (_MW - 1.0) + gx28 * ((bx2 - bx1) * (_MW - 1.0))
            y0 = jnp.floor(ys)
            wy = ys - y0
            x0 = jnp.floor(xs)
            wx = xs - x0
            w_y = ((irow56 == y0).astype(f32) * (1.0 - wy) +
                   (irow56 == y0 + 1.0).astype(f32) * wy)
            w_xt = ((icol56 == x0).astype(f32) * (1.0 - wx) +
                    (icol56 == x0 + 1.0).astype(f32) * wx)
            mrow = msk_ref[pl.ds(g * _MH, _MH), b * _MW:(b + 1) * _MW]
            t = jnp.dot(mrow, w_xt, preferred_element_type=f32,
                        precision=jax.lax.Precision.HIGHEST)
            crop = jnp.dot(w_y, t, preferred_element_type=f32,
                           precision=jax.lax.Precision.HIGHEST)
            out = jnp.round(crop) * vv
            msko_ref[pl.ds(k * 32, _MKH), b * _MKW:(b + 1) * _MKW] = out
        return carry

    jax.lax.fori_loop(0, _POS, mask_step, 0)


@jax.jit
def kernel(proposals, prior_class_ids, prior_boxes, prior_masks):
    b = proposals.shape[0]
    f32 = jnp.float32
    p_pad = jnp.transpose(
        jnp.pad(proposals, ((0, 0), (0, _NPP - _NP), (0, 0))),
        (1, 0, 2)).reshape(_NPP, b * 4)
    gt_t = jnp.pad(jnp.transpose(prior_boxes, (0, 2, 1)),
                   ((0, 0), (0, 0), (0, _GTL - _MAXGT)))
    gt_r = jnp.pad(prior_boxes, ((0, 0), (0, _GTL - _MAXGT), (0, 0)))
    cls_f = jnp.pad(prior_class_ids.astype(f32),
                    ((0, 0), (0, _GTL - _MAXGT)))[:, None, :]
    msk_t = jnp.pad(jnp.transpose(prior_masks, (3, 1, 0, 2)),
                    ((0, 104 - _MAXGT), (0, 0), (0, 0), (0, 0))
                    ).reshape(104 * _MH, b * _MW)

    def full(*shape):
        nd = len(shape)
        return pl.BlockSpec(shape, lambda: (0,) * nd)

    rois, cls_o, deltas, masks = pl.pallas_call(
        _body,
        grid=(),
        in_specs=[
            full(_NPP, b * 4),
            full(b, 4, _GTL),
            full(b, _GTL, 4),
            full(b, 1, _GTL),
            full(104 * _MH, b * _MW),
        ],
        out_specs=[
            full(b, _TRAIN, 4),
            full(b, _TRAIN, 1),
            full(b, _TRAIN, 4),
            full(_TRAIN * 32, b * _MKW),
        ],
        out_shape=[
            jax.ShapeDtypeStruct((b, _TRAIN, 4), f32),
            jax.ShapeDtypeStruct((b, _TRAIN, 1), jnp.int32),
            jax.ShapeDtypeStruct((b, _TRAIN, 4), f32),
            jax.ShapeDtypeStruct((_TRAIN * 32, b * _MKW), f32),
        ],
        scratch_shapes=[
            pltpu.VMEM((b * 40, 128), f32),
            pltpu.VMEM((b * 40, 128), f32),
            pltpu.VMEM((_NPP, b), f32),
            pltpu.VMEM((b * _TRAIN, 4), f32),
            pltpu.VMEM((b * _TRAIN, 1), f32),
            pltpu.VMEM((b * _POSP, 1), f32),
            pltpu.VMEM((b * _POSP, 4), f32),
        ],
    )(p_pad, gt_t, gt_r, cls_f, msk_t)
    masks = jnp.transpose(
        masks.reshape(_TRAIN, 32, b, _MKW), (2, 0, 1, 3))[:, :, :_MKH, :]
    return rois, cls_o[:, :, 0], deltas, masks
